# baseline (speedup 1.0000x reference)
import functools

import jax
import jax.numpy as jnp
from jax import lax
from jax.experimental import pallas as pl
from jax.experimental.pallas import tpu as pltpu
from jax.experimental.pallas import tpu_sc as plsc

V = 100000
E = 64
H = 256
L = 64
B = 1024
T = 200

NC = 2
NS = 16
NW = NC * NS
CHUNK = 128
NROWS = (T * B) // CHUNK
ROWS_W = NROWS // NW
PER_W = ROWS_W * CHUNK

NBC = 2
BC = B // NBC

@functools.cache
def _make_sc_gather():
    mesh = plsc.VectorSubcoreMesh(
        core_axis_name="c", subcore_axis_name="s", num_cores=NC, num_subcores=NS
    )

    @functools.partial(
        pl.kernel,
        out_type=jax.ShapeDtypeStruct((T * B, E), jnp.float32),
        mesh=mesh,
        scratch_types=[
            pltpu.VMEM((ROWS_W, CHUNK), jnp.int32),
            pltpu.VMEM((CHUNK, E), jnp.float32),
            pltpu.SemaphoreType.DMA,
        ],
        compiler_params=pltpu.CompilerParams(use_tc_tiling_on_sc=False),
    )
    def _sc_gather(emb_hbm, idx_hbm, out_hbm, idx_v, rows_v, sem):
        wid = lax.axis_index("s") * NC + lax.axis_index("c")
        pltpu.sync_copy(idx_hbm.at[wid], idx_v)
        base = wid * PER_W

        def body(j, carry):
            pltpu.async_copy(emb_hbm.at[idx_v.at[j]], rows_v, sem).wait()
            pltpu.sync_copy(rows_v, out_hbm.at[pl.ds(base + j * CHUNK, CHUNK)])
            return carry

        lax.fori_loop(0, ROWS_W, body, 0)

    return _sc_gather


def _rnn_body(x_ref, wih_ref, whh_ref, brz_ref, bihn_ref, bhhn_ref,
              wihb_ref, brzb_ref, bihbn_ref, bhhbn_ref,
              wmu_ref, bmu_ref, wlv_ref, blv_ref,
              mu_ref, lv_ref, h_scr):
    t = pl.program_id(0)

    @pl.when(t == 0)
    def _():
        h_scr[...] = jnp.zeros_like(h_scr)

    hn_chunks = []
    for c in range(NBC):
        lo = c * BC
        x = x_ref[0, lo:lo + BC, :]
        h = h_scr[lo:lo + BC, :]
        gi = jnp.dot(x.astype(jnp.bfloat16), wih_ref[...],
                     preferred_element_type=jnp.float32)
        gh = jnp.dot(h.astype(jnp.bfloat16), whh_ref[...],
                     preferred_element_type=jnp.float32)
        grz = gi[:, :2 * H] + gh[:, :2 * H] + brz_ref[...]
        r = 0.5 + 0.5 * jnp.tanh(grz[:, :H])
        z = 0.5 + 0.5 * jnp.tanh(grz[:, H:])
        n = jnp.tanh((gi[:, 2 * H:] + bihn_ref[...])
                     + r * (gh[:, 2 * H:] + bhhn_ref[...]))
        hn = n + z * (h - n)
        h_scr[lo:lo + BC, :] = hn
        hn_chunks.append(hn)

    @pl.when(t == T - 1)
    def _():
        for c in range(NBC):
            lo = c * BC
            x = x_ref[0, lo:lo + BC, :]
            gib = jnp.dot(x.astype(jnp.bfloat16), wihb_ref[...],
                          preferred_element_type=jnp.float32)
            grzb = gib[:, :2 * H] + brzb_ref[...]
            rb = 0.5 + 0.5 * jnp.tanh(grzb[:, :H])
            zb = 0.5 - 0.5 * jnp.tanh(grzb[:, H:])
            nb = jnp.tanh((gib[:, 2 * H:] + bihbn_ref[...]) + rb * bhhbn_ref[...])
            out = hn_chunks[c] + zb * nb
            mu_ref[lo:lo + BC, :] = (
                jnp.dot(out, wmu_ref[...], preferred_element_type=jnp.float32)
                + bmu_ref[...])
            lv_ref[lo:lo + BC, :] = (
                jnp.dot(out, wlv_ref[...], preferred_element_type=jnp.float32)
                + blv_ref[...])


_FULL2 = lambda t: (0, 0)

_rnn_call = pl.pallas_call(
    _rnn_body,
    grid=(T,),
    in_specs=[
        pl.BlockSpec((1, B, E), lambda t: (t, 0, 0)),
        pl.BlockSpec((E, 3 * H), _FULL2),
        pl.BlockSpec((H, 3 * H), _FULL2),
        pl.BlockSpec((1, 2 * H), _FULL2),
        pl.BlockSpec((1, H), _FULL2),
        pl.BlockSpec((1, H), _FULL2),
        pl.BlockSpec((E, 3 * H), _FULL2),
        pl.BlockSpec((1, 2 * H), _FULL2),
        pl.BlockSpec((1, H), _FULL2),
        pl.BlockSpec((1, H), _FULL2),
        pl.BlockSpec((H, L), _FULL2),
        pl.BlockSpec((1, L), _FULL2),
        pl.BlockSpec((H, L), _FULL2),
        pl.BlockSpec((1, L), _FULL2),
    ],
    out_specs=[pl.BlockSpec((B, L), _FULL2), pl.BlockSpec((B, L), _FULL2)],
    out_shape=[jax.ShapeDtypeStruct((B, L), jnp.float32)] * 2,
    scratch_shapes=[pltpu.VMEM((B, H), jnp.float32)],
)


def kernel(inputs, emb, W_ih_f, W_hh_f, b_ih_f, b_hh_f,
           W_ih_b, W_hh_b, b_ih_b, b_hh_b, W_mu, b_mu, W_lv, b_lv):
    idx = inputs.astype(jnp.int32).T.reshape(NW, ROWS_W, CHUNK)
    x_flat = _make_sc_gather()(emb, idx)
    x3 = x_flat.reshape(T, B, E)

    scale = jnp.concatenate(
        [jnp.full((2 * H,), 0.5, jnp.float32), jnp.ones((H,), jnp.float32)])
    wih_s = (W_ih_f.T * scale).astype(jnp.bfloat16)
    whh_s = (W_hh_f.T * scale).astype(jnp.bfloat16)
    brz = (0.5 * (b_ih_f[:2 * H] + b_hh_f[:2 * H])).reshape(1, -1)
    bihn = b_ih_f[2 * H:].reshape(1, -1)
    bhhn = b_hh_f[2 * H:].reshape(1, -1)
    wihb_s = (W_ih_b.T * scale).astype(jnp.bfloat16)
    brzb = (0.5 * (b_ih_b[:2 * H] + b_hh_b[:2 * H])).reshape(1, -1)
    bihbn = b_ih_b[2 * H:].reshape(1, -1)
    bhhbn = b_hh_b[2 * H:].reshape(1, -1)

    mu, lv = _rnn_call(
        x3,
        wih_s, whh_s, brz, bihn, bhhn,
        wihb_s, brzb, bihbn, bhhbn,
        W_mu.T, b_mu.reshape(1, -1), W_lv.T, b_lv.reshape(1, -1),
    )
    return (mu, lv)


# speedup vs baseline: 1.1661x; 1.1661x over previous
import functools

import jax
import jax.numpy as jnp
from jax import lax
from jax.experimental import pallas as pl
from jax.experimental.pallas import tpu as pltpu
from jax.experimental.pallas import tpu_sc as plsc

V = 100000
E = 64
H = 256
L = 64
B = 1024
T = 200
B2 = B // 2

NC = 2
NS = 16
NW = NC * NS
CHUNK = 128
NROWS = (T * B) // CHUNK
ROWS_W = NROWS // NW
PER_W = ROWS_W * CHUNK


@functools.cache
def _make_sc_gather():
    mesh = plsc.VectorSubcoreMesh(
        core_axis_name="c", subcore_axis_name="s", num_cores=NC, num_subcores=NS
    )

    @functools.partial(
        pl.kernel,
        out_type=jax.ShapeDtypeStruct((T * B2, 2 * E), jnp.float32),
        mesh=mesh,
        scratch_types=[
            pltpu.VMEM((ROWS_W, CHUNK), jnp.int32),
            pltpu.VMEM((CHUNK // 2, E), jnp.float32),
            pltpu.VMEM((CHUNK // 2, E), jnp.float32),
            pltpu.SemaphoreType.DMA,
        ],
        compiler_params=pltpu.CompilerParams(use_tc_tiling_on_sc=False),
    )
    def _sc_gather(emb_hbm, idx_hbm, out_hbm, idx_v, rows_a, rows_b, sem):
        wid = lax.axis_index("s") * NC + lax.axis_index("c")
        pltpu.sync_copy(idx_hbm.at[wid], idx_v)
        base2 = wid * (PER_W // 2)

        def body(j, carry):
            ca = pltpu.async_copy(emb_hbm.at[idx_v.at[j, pl.ds(0, CHUNK // 2)]],
                                  rows_a, sem)
            cb = pltpu.async_copy(emb_hbm.at[idx_v.at[j, pl.ds(CHUNK // 2, CHUNK // 2)]],
                                  rows_b, sem)
            ca.wait()
            cb.wait()
            row0 = base2 + j * (CHUNK // 2)
            pltpu.sync_copy(rows_a,
                            out_hbm.at[pl.ds(row0, CHUNK // 2), pl.ds(0, E)])
            pltpu.sync_copy(rows_b,
                            out_hbm.at[pl.ds(row0, CHUNK // 2), pl.ds(E, E)])
            return carry

        lax.fori_loop(0, ROWS_W, body, 0)

    return _sc_gather


def _rnn_body(x_ref, wih2_ref, whh_ref, brz_ref, bihn_ref, bhhn_ref,
              wihb2_ref, brzb_ref, bihbn_ref, bhhbn_ref,
              wmu_ref, bmu_ref, wlv_ref, blv_ref,
              mu_ref, lv_ref, h_scr):
    t = pl.program_id(0)

    @pl.when(t == 0)
    def _():
        h_scr[...] = jnp.zeros_like(h_scr)

    xp = x_ref[0]
    gi2 = jnp.dot(xp.astype(jnp.bfloat16), wih2_ref[...],
                  preferred_element_type=jnp.float32)
    hn_halves = []
    for c in range(2):
        h = h_scr[:, c * H:(c + 1) * H]
        gi = gi2[:, c * 3 * H:(c + 1) * 3 * H]
        gh = jnp.dot(h.astype(jnp.bfloat16), whh_ref[...],
                     preferred_element_type=jnp.float32)
        grz = gi[:, :2 * H] + gh[:, :2 * H] + brz_ref[...]
        r = 0.5 + 0.5 * jnp.tanh(grz[:, :H])
        z = 0.5 + 0.5 * jnp.tanh(grz[:, H:])
        n = jnp.tanh((gi[:, 2 * H:] + bihn_ref[...])
                     + r * (gh[:, 2 * H:] + bhhn_ref[...]))
        hn = n + z * (h - n)
        h_scr[:, c * H:(c + 1) * H] = hn
        hn_halves.append(hn)

    @pl.when(t == T - 1)
    def _():
        gib2 = jnp.dot(xp.astype(jnp.bfloat16), wihb2_ref[...],
                       preferred_element_type=jnp.float32)
        for c in range(2):
            gib = gib2[:, c * 3 * H:(c + 1) * 3 * H]
            grzb = gib[:, :2 * H] + brzb_ref[...]
            rb = 0.5 + 0.5 * jnp.tanh(grzb[:, :H])
            zb = 0.5 - 0.5 * jnp.tanh(grzb[:, H:])
            nb = jnp.tanh((gib[:, 2 * H:] + bihbn_ref[...]) + rb * bhhbn_ref[...])
            out = hn_halves[c] + zb * nb
            mu_ref[c * B2:(c + 1) * B2, :] = (
                jnp.dot(out, wmu_ref[...], preferred_element_type=jnp.float32)
                + bmu_ref[...])
            lv_ref[c * B2:(c + 1) * B2, :] = (
                jnp.dot(out, wlv_ref[...], preferred_element_type=jnp.float32)
                + blv_ref[...])


_FULL2 = lambda t: (0, 0)

_rnn_call = pl.pallas_call(
    _rnn_body,
    grid=(T,),
    in_specs=[
        pl.BlockSpec((1, B2, 2 * E), lambda t: (t, 0, 0)),
        pl.BlockSpec((2 * E, 6 * H), _FULL2),
        pl.BlockSpec((H, 3 * H), _FULL2),
        pl.BlockSpec((1, 2 * H), _FULL2),
        pl.BlockSpec((1, H), _FULL2),
        pl.BlockSpec((1, H), _FULL2),
        pl.BlockSpec((2 * E, 6 * H), _FULL2),
        pl.BlockSpec((1, 2 * H), _FULL2),
        pl.BlockSpec((1, H), _FULL2),
        pl.BlockSpec((1, H), _FULL2),
        pl.BlockSpec((H, L), _FULL2),
        pl.BlockSpec((1, L), _FULL2),
        pl.BlockSpec((H, L), _FULL2),
        pl.BlockSpec((1, L), _FULL2),
    ],
    out_specs=[pl.BlockSpec((B, L), _FULL2), pl.BlockSpec((B, L), _FULL2)],
    out_shape=[jax.ShapeDtypeStruct((B, L), jnp.float32)] * 2,
    scratch_shapes=[pltpu.VMEM((B2, 2 * H), jnp.float32)],
)


def _blockdiag2(w):
    zero = jnp.zeros_like(w)
    return jnp.concatenate(
        [jnp.concatenate([w, zero], axis=1),
         jnp.concatenate([zero, w], axis=1)], axis=0)


def kernel(inputs, emb, W_ih_f, W_hh_f, b_ih_f, b_hh_f,
           W_ih_b, W_hh_b, b_ih_b, b_hh_b, W_mu, b_mu, W_lv, b_lv):
    ii = inputs.astype(jnp.int32)
    idx_a = ii[:B2].T.reshape(T * B2 // (CHUNK // 2), CHUNK // 2)
    idx_b = ii[B2:].T.reshape(T * B2 // (CHUNK // 2), CHUNK // 2)
    idx = jnp.concatenate([idx_a, idx_b], axis=1).reshape(NW, ROWS_W, CHUNK)
    xp_flat = _make_sc_gather()(emb, idx)
    x3 = xp_flat.reshape(T, B2, 2 * E)

    scale = jnp.concatenate(
        [jnp.full((2 * H,), 0.5, jnp.float32), jnp.ones((H,), jnp.float32)])
    wih2 = _blockdiag2(W_ih_f.T * scale).astype(jnp.bfloat16)
    whh_s = (W_hh_f.T * scale).astype(jnp.bfloat16)
    brz = (0.5 * (b_ih_f[:2 * H] + b_hh_f[:2 * H])).reshape(1, -1)
    bihn = b_ih_f[2 * H:].reshape(1, -1)
    bhhn = b_hh_f[2 * H:].reshape(1, -1)
    wihb2 = _blockdiag2(W_ih_b.T * scale).astype(jnp.bfloat16)
    brzb = (0.5 * (b_ih_b[:2 * H] + b_hh_b[:2 * H])).reshape(1, -1)
    bihbn = b_ih_b[2 * H:].reshape(1, -1)
    bhhbn = b_hh_b[2 * H:].reshape(1, -1)

    mu, lv = _rnn_call(
        x3,
        wih2, whh_s, brz, bihn, bhhn,
        wihb2, brzb, bihbn, bhhbn,
        W_mu.T, b_mu.reshape(1, -1), W_lv.T, b_lv.reshape(1, -1),
    )
    return (mu, lv)


# speedup vs baseline: 1.3230x; 1.1345x over previous
import functools

import jax
import jax.numpy as jnp
from jax import lax
from jax.experimental import pallas as pl
from jax.experimental.pallas import tpu as pltpu
from jax.experimental.pallas import tpu_sc as plsc

V = 100000
E = 64
H = 256
L = 64
B = 1024
T = 200
B2 = B // 2

NCH = 5
TCH = T // NCH

NC = 2
NS = 16
NW = NC * NS
CHUNK = 128
NROWS = (T * B) // CHUNK
ROWS_C = NROWS // NCH
ROWS_W = ROWS_C // NW
OUT_C = TCH * B2


@functools.cache
def _make_sc_gather():
    mesh = plsc.VectorSubcoreMesh(
        core_axis_name="c", subcore_axis_name="s", num_cores=NC, num_subcores=NS
    )

    @functools.partial(
        pl.kernel,
        out_type=jax.ShapeDtypeStruct((OUT_C, 2 * E), jnp.float32),
        mesh=mesh,
        scratch_types=[
            pltpu.VMEM((ROWS_W, CHUNK), jnp.int32),
            pltpu.VMEM((CHUNK // 2, E), jnp.float32),
            pltpu.VMEM((CHUNK // 2, E), jnp.float32),
            pltpu.SemaphoreType.DMA,
        ],
        compiler_params=pltpu.CompilerParams(use_tc_tiling_on_sc=False),
    )
    def _sc_gather(emb_hbm, idx_hbm, out_hbm, idx_v, rows_a, rows_b, sem):
        wid = lax.axis_index("s") * NC + lax.axis_index("c")
        pltpu.sync_copy(idx_hbm.at[wid], idx_v)
        base2 = wid * ROWS_W * (CHUNK // 2)

        def body(j, carry):
            ca = pltpu.async_copy(emb_hbm.at[idx_v.at[j, pl.ds(0, CHUNK // 2)]],
                                  rows_a, sem)
            cb = pltpu.async_copy(emb_hbm.at[idx_v.at[j, pl.ds(CHUNK // 2, CHUNK // 2)]],
                                  rows_b, sem)
            ca.wait()
            cb.wait()
            row0 = base2 + j * (CHUNK // 2)
            pltpu.sync_copy(rows_a,
                            out_hbm.at[pl.ds(row0, CHUNK // 2), pl.ds(0, E)])
            pltpu.sync_copy(rows_b,
                            out_hbm.at[pl.ds(row0, CHUNK // 2), pl.ds(E, E)])
            return carry

        lax.fori_loop(0, ROWS_W, body, 0)

    return _sc_gather


def _gru_steps(t, x_ref, hin_ref, wih2_ref, whh_ref, brz_ref, bihn_ref,
               bhhn_ref, h_scr):
    @pl.when(t == 0)
    def _():
        h_scr[...] = hin_ref[...]

    xp = x_ref[0]
    gi2 = jnp.dot(xp.astype(jnp.bfloat16), wih2_ref[...],
                  preferred_element_type=jnp.float32)
    hn_halves = []
    for c in range(2):
        h = h_scr[:, c * H:(c + 1) * H]
        gi = gi2[:, c * 3 * H:(c + 1) * 3 * H]
        gh = jnp.dot(h.astype(jnp.bfloat16), whh_ref[...],
                     preferred_element_type=jnp.float32)
        grz = gi[:, :2 * H] + gh[:, :2 * H] + brz_ref[...]
        r = 0.5 + 0.5 * jnp.tanh(grz[:, :H])
        z = 0.5 + 0.5 * jnp.tanh(grz[:, H:])
        n = jnp.tanh((gi[:, 2 * H:] + bihn_ref[...])
                     + r * (gh[:, 2 * H:] + bhhn_ref[...]))
        hn = n + z * (h - n)
        h_scr[:, c * H:(c + 1) * H] = hn
        hn_halves.append(hn)
    return xp, hn_halves


def _rnn_mid_body(x_ref, hin_ref, wih2_ref, whh_ref, brz_ref, bihn_ref,
                  bhhn_ref, hout_ref, h_scr):
    t = pl.program_id(0)
    _gru_steps(t, x_ref, hin_ref, wih2_ref, whh_ref, brz_ref, bihn_ref,
               bhhn_ref, h_scr)

    @pl.when(t == TCH - 1)
    def _():
        hout_ref[...] = h_scr[...]


def _rnn_fin_body(x_ref, hin_ref, wih2_ref, whh_ref, brz_ref, bihn_ref,
                  bhhn_ref, wihb2_ref, brzb_ref, bihbn_ref, bhhbn_ref,
                  wmu_ref, bmu_ref, wlv_ref, blv_ref,
                  mu_ref, lv_ref, h_scr):
    t = pl.program_id(0)
    xp, hn_halves = _gru_steps(t, x_ref, hin_ref, wih2_ref, whh_ref, brz_ref,
                               bihn_ref, bhhn_ref, h_scr)

    @pl.when(t == TCH - 1)
    def _():
        gib2 = jnp.dot(xp.astype(jnp.bfloat16), wihb2_ref[...],
                       preferred_element_type=jnp.float32)
        for c in range(2):
            gib = gib2[:, c * 3 * H:(c + 1) * 3 * H]
            grzb = gib[:, :2 * H] + brzb_ref[...]
            rb = 0.5 + 0.5 * jnp.tanh(grzb[:, :H])
            zb = 0.5 - 0.5 * jnp.tanh(grzb[:, H:])
            nb = jnp.tanh((gib[:, 2 * H:] + bihbn_ref[...]) + rb * bhhbn_ref[...])
            out = hn_halves[c] + zb * nb
            mu_ref[c * B2:(c + 1) * B2, :] = (
                jnp.dot(out, wmu_ref[...], preferred_element_type=jnp.float32)
                + bmu_ref[...])
            lv_ref[c * B2:(c + 1) * B2, :] = (
                jnp.dot(out, wlv_ref[...], preferred_element_type=jnp.float32)
                + blv_ref[...])


_FULL2 = lambda t: (0, 0)

_X_SPEC = pl.BlockSpec((1, B2, 2 * E), lambda t: (t, 0, 0))
_H_SPEC = pl.BlockSpec((B2, 2 * H), _FULL2)
_FWD_W_SPECS = [
    _H_SPEC,
    pl.BlockSpec((2 * E, 6 * H), _FULL2),
    pl.BlockSpec((H, 3 * H), _FULL2),
    pl.BlockSpec((1, 2 * H), _FULL2),
    pl.BlockSpec((1, H), _FULL2),
    pl.BlockSpec((1, H), _FULL2),
]

_rnn_mid = pl.pallas_call(
    _rnn_mid_body,
    grid=(TCH,),
    in_specs=[_X_SPEC] + _FWD_W_SPECS,
    out_specs=[_H_SPEC],
    out_shape=[jax.ShapeDtypeStruct((B2, 2 * H), jnp.float32)],
    scratch_shapes=[pltpu.VMEM((B2, 2 * H), jnp.float32)],
)

_rnn_fin = pl.pallas_call(
    _rnn_fin_body,
    grid=(TCH,),
    in_specs=[_X_SPEC] + _FWD_W_SPECS + [
        pl.BlockSpec((2 * E, 6 * H), _FULL2),
        pl.BlockSpec((1, 2 * H), _FULL2),
        pl.BlockSpec((1, H), _FULL2),
        pl.BlockSpec((1, H), _FULL2),
        pl.BlockSpec((H, L), _FULL2),
        pl.BlockSpec((1, L), _FULL2),
        pl.BlockSpec((H, L), _FULL2),
        pl.BlockSpec((1, L), _FULL2),
    ],
    out_specs=[pl.BlockSpec((B, L), _FULL2), pl.BlockSpec((B, L), _FULL2)],
    out_shape=[jax.ShapeDtypeStruct((B, L), jnp.float32)] * 2,
    scratch_shapes=[pltpu.VMEM((B2, 2 * H), jnp.float32)],
)


def _blockdiag2(w):
    zero = jnp.zeros_like(w)
    return jnp.concatenate(
        [jnp.concatenate([w, zero], axis=1),
         jnp.concatenate([zero, w], axis=1)], axis=0)


def kernel(inputs, emb, W_ih_f, W_hh_f, b_ih_f, b_hh_f,
           W_ih_b, W_hh_b, b_ih_b, b_hh_b, W_mu, b_mu, W_lv, b_lv):
    ii = inputs.astype(jnp.int32)
    idx_a = ii[:B2].T.reshape(NROWS, CHUNK // 2)
    idx_b = ii[B2:].T.reshape(NROWS, CHUNK // 2)
    idx = jnp.concatenate([idx_a, idx_b], axis=1)

    gather = _make_sc_gather()
    xs = []
    for c in range(NCH):
        idx_c = idx[c * ROWS_C:(c + 1) * ROWS_C].reshape(NW, ROWS_W, CHUNK)
        xs.append(gather(emb, idx_c).reshape(TCH, B2, 2 * E))

    scale = jnp.concatenate(
        [jnp.full((2 * H,), 0.5, jnp.float32), jnp.ones((H,), jnp.float32)])
    wih2 = _blockdiag2(W_ih_f.T * scale).astype(jnp.bfloat16)
    whh_s = (W_hh_f.T * scale).astype(jnp.bfloat16)
    brz = (0.5 * (b_ih_f[:2 * H] + b_hh_f[:2 * H])).reshape(1, -1)
    bihn = b_ih_f[2 * H:].reshape(1, -1)
    bhhn = b_hh_f[2 * H:].reshape(1, -1)
    wihb2 = _blockdiag2(W_ih_b.T * scale).astype(jnp.bfloat16)
    brzb = (0.5 * (b_ih_b[:2 * H] + b_hh_b[:2 * H])).reshape(1, -1)
    bihbn = b_ih_b[2 * H:].reshape(1, -1)
    bhhbn = b_hh_b[2 * H:].reshape(1, -1)

    fwd_w = (wih2, whh_s, brz, bihn, bhhn)
    h = jnp.zeros((B2, 2 * H), jnp.float32)
    for c in range(NCH - 1):
        (h,) = _rnn_mid(xs[c], h, *fwd_w)
    mu, lv = _rnn_fin(
        xs[NCH - 1], h, *fwd_w,
        wihb2, brzb, bihbn, bhhbn,
        W_mu.T, b_mu.reshape(1, -1), W_lv.T, b_lv.reshape(1, -1),
    )
    return (mu, lv)


# speedup vs baseline: 1.3347x; 1.0088x over previous
import functools

import jax
import jax.numpy as jnp
from jax import lax
from jax.experimental import pallas as pl
from jax.experimental.pallas import tpu as pltpu
from jax.experimental.pallas import tpu_sc as plsc

V = 100000
E = 64
H = 256
L = 64
B = 1024
T = 200
B2 = B // 2

NCH = 5
TCH = T // NCH

NC = 2
NS = 16
NW = NC * NS
CHUNK = 128
NROWS = (T * B) // CHUNK
ROWS_C = NROWS // NCH
ROWS_W = ROWS_C // NW
OUT_C = TCH * B2


@functools.cache
def _make_sc_gather():
    mesh = plsc.VectorSubcoreMesh(
        core_axis_name="c", subcore_axis_name="s", num_cores=NC, num_subcores=NS
    )

    @functools.partial(
        pl.kernel,
        out_type=jax.ShapeDtypeStruct((OUT_C, 2 * E), jnp.float32),
        mesh=mesh,
        scratch_types=[
            pltpu.VMEM((ROWS_W, CHUNK), jnp.int32),
            pltpu.VMEM((CHUNK // 2, E), jnp.float32),
            pltpu.VMEM((CHUNK // 2, E), jnp.float32),
            pltpu.SemaphoreType.DMA,
        ],
        compiler_params=pltpu.CompilerParams(use_tc_tiling_on_sc=False),
    )
    def _sc_gather(emb_hbm, idx_hbm, out_hbm, idx_v, rows_a, rows_b, sem):
        wid = lax.axis_index("s") * NC + lax.axis_index("c")
        pltpu.sync_copy(idx_hbm.at[wid], idx_v)
        base2 = wid * ROWS_W * (CHUNK // 2)

        def body(j, carry):
            ca = pltpu.async_copy(emb_hbm.at[idx_v.at[j, pl.ds(0, CHUNK // 2)]],
                                  rows_a, sem)
            cb = pltpu.async_copy(emb_hbm.at[idx_v.at[j, pl.ds(CHUNK // 2, CHUNK // 2)]],
                                  rows_b, sem)
            ca.wait()
            cb.wait()
            row0 = base2 + j * (CHUNK // 2)
            pltpu.sync_copy(rows_a,
                            out_hbm.at[pl.ds(row0, CHUNK // 2), pl.ds(0, E)])
            pltpu.sync_copy(rows_b,
                            out_hbm.at[pl.ds(row0, CHUNK // 2), pl.ds(E, E)])
            return carry

        lax.fori_loop(0, ROWS_W, body, 0)

    return _sc_gather


def _gru_steps(t, x_ref, hin_ref, wih2_ref, whh_ref, brz_ref, bihn_ref,
               bhhn_ref, h_scr, h16_scr):
    @pl.when(t == 0)
    def _():
        h_scr[...] = hin_ref[...]
        h16_scr[...] = hin_ref[...].astype(jnp.bfloat16)

    xp = x_ref[0]
    gi2 = jnp.dot(xp.astype(jnp.bfloat16), wih2_ref[...],
                  preferred_element_type=jnp.float32)
    hn_halves = []
    for c in range(2):
        h = h_scr[:, c * H:(c + 1) * H]
        gi = gi2[:, c * 3 * H:(c + 1) * 3 * H]
        gh = jnp.dot(h16_scr[:, c * H:(c + 1) * H], whh_ref[...],
                     preferred_element_type=jnp.float32)
        grz = gi[:, :2 * H] + gh[:, :2 * H] + brz_ref[...]
        r = 0.5 + 0.5 * jnp.tanh(grz[:, :H])
        z = 0.5 + 0.5 * jnp.tanh(grz[:, H:])
        n = jnp.tanh((gi[:, 2 * H:] + bihn_ref[...])
                     + r * (gh[:, 2 * H:] + bhhn_ref[...]))
        hn = n + z * (h - n)
        h_scr[:, c * H:(c + 1) * H] = hn
        h16_scr[:, c * H:(c + 1) * H] = hn.astype(jnp.bfloat16)
        hn_halves.append(hn)
    return xp, hn_halves


def _rnn_mid_body(x_ref, hin_ref, wih2_ref, whh_ref, brz_ref, bihn_ref,
                  bhhn_ref, hout_ref, h_scr, h16_scr):
    t = pl.program_id(0)
    _gru_steps(t, x_ref, hin_ref, wih2_ref, whh_ref, brz_ref, bihn_ref,
               bhhn_ref, h_scr, h16_scr)

    @pl.when(t == TCH - 1)
    def _():
        hout_ref[...] = h_scr[...]


def _rnn_fin_body(x_ref, hin_ref, wih2_ref, whh_ref, brz_ref, bihn_ref,
                  bhhn_ref, wihb2_ref, brzb_ref, bihbn_ref, bhhbn_ref,
                  wmu_ref, bmu_ref, wlv_ref, blv_ref,
                  mu_ref, lv_ref, h_scr, h16_scr):
    t = pl.program_id(0)
    xp, hn_halves = _gru_steps(t, x_ref, hin_ref, wih2_ref, whh_ref, brz_ref,
                               bihn_ref, bhhn_ref, h_scr, h16_scr)

    @pl.when(t == TCH - 1)
    def _():
        gib2 = jnp.dot(xp.astype(jnp.bfloat16), wihb2_ref[...],
                       preferred_element_type=jnp.float32)
        for c in range(2):
            gib = gib2[:, c * 3 * H:(c + 1) * 3 * H]
            grzb = gib[:, :2 * H] + brzb_ref[...]
            rb = 0.5 + 0.5 * jnp.tanh(grzb[:, :H])
            zb = 0.5 - 0.5 * jnp.tanh(grzb[:, H:])
            nb = jnp.tanh((gib[:, 2 * H:] + bihbn_ref[...]) + rb * bhhbn_ref[...])
            out = hn_halves[c] + zb * nb
            mu_ref[c * B2:(c + 1) * B2, :] = (
                jnp.dot(out, wmu_ref[...], preferred_element_type=jnp.float32)
                + bmu_ref[...])
            lv_ref[c * B2:(c + 1) * B2, :] = (
                jnp.dot(out, wlv_ref[...], preferred_element_type=jnp.float32)
                + blv_ref[...])


_FULL2 = lambda t: (0, 0)

_X_SPEC = pl.BlockSpec((1, B2, 2 * E), lambda t: (t, 0, 0))
_H_SPEC = pl.BlockSpec((B2, 2 * H), _FULL2)
_FWD_W_SPECS = [
    _H_SPEC,
    pl.BlockSpec((2 * E, 6 * H), _FULL2),
    pl.BlockSpec((H, 3 * H), _FULL2),
    pl.BlockSpec((1, 2 * H), _FULL2),
    pl.BlockSpec((1, H), _FULL2),
    pl.BlockSpec((1, H), _FULL2),
]

_rnn_mid = pl.pallas_call(
    _rnn_mid_body,
    grid=(TCH,),
    in_specs=[_X_SPEC] + _FWD_W_SPECS,
    out_specs=[_H_SPEC],
    out_shape=[jax.ShapeDtypeStruct((B2, 2 * H), jnp.float32)],
    scratch_shapes=[pltpu.VMEM((B2, 2 * H), jnp.float32),
                    pltpu.VMEM((B2, 2 * H), jnp.bfloat16)],
)

_rnn_fin = pl.pallas_call(
    _rnn_fin_body,
    grid=(TCH,),
    in_specs=[_X_SPEC] + _FWD_W_SPECS + [
        pl.BlockSpec((2 * E, 6 * H), _FULL2),
        pl.BlockSpec((1, 2 * H), _FULL2),
        pl.BlockSpec((1, H), _FULL2),
        pl.BlockSpec((1, H), _FULL2),
        pl.BlockSpec((H, L), _FULL2),
        pl.BlockSpec((1, L), _FULL2),
        pl.BlockSpec((H, L), _FULL2),
        pl.BlockSpec((1, L), _FULL2),
    ],
    out_specs=[pl.BlockSpec((B, L), _FULL2), pl.BlockSpec((B, L), _FULL2)],
    out_shape=[jax.ShapeDtypeStruct((B, L), jnp.float32)] * 2,
    scratch_shapes=[pltpu.VMEM((B2, 2 * H), jnp.float32),
                    pltpu.VMEM((B2, 2 * H), jnp.bfloat16)],
)


def _blockdiag2(w):
    zero = jnp.zeros_like(w)
    return jnp.concatenate(
        [jnp.concatenate([w, zero], axis=1),
         jnp.concatenate([zero, w], axis=1)], axis=0)


def kernel(inputs, emb, W_ih_f, W_hh_f, b_ih_f, b_hh_f,
           W_ih_b, W_hh_b, b_ih_b, b_hh_b, W_mu, b_mu, W_lv, b_lv):
    ii = inputs.astype(jnp.int32)
    idx_a = ii[:B2].T.reshape(NROWS, CHUNK // 2)
    idx_b = ii[B2:].T.reshape(NROWS, CHUNK // 2)
    idx = jnp.concatenate([idx_a, idx_b], axis=1)

    gather = _make_sc_gather()
    xs = []
    for c in range(NCH):
        idx_c = idx[c * ROWS_C:(c + 1) * ROWS_C].reshape(NW, ROWS_W, CHUNK)
        xs.append(gather(emb, idx_c).reshape(TCH, B2, 2 * E))

    scale = jnp.concatenate(
        [jnp.full((2 * H,), 0.5, jnp.float32), jnp.ones((H,), jnp.float32)])
    wih2 = _blockdiag2(W_ih_f.T * scale).astype(jnp.bfloat16)
    whh_s = (W_hh_f.T * scale).astype(jnp.bfloat16)
    brz = (0.5 * (b_ih_f[:2 * H] + b_hh_f[:2 * H])).reshape(1, -1)
    bihn = b_ih_f[2 * H:].reshape(1, -1)
    bhhn = b_hh_f[2 * H:].reshape(1, -1)
    wihb2 = _blockdiag2(W_ih_b.T * scale).astype(jnp.bfloat16)
    brzb = (0.5 * (b_ih_b[:2 * H] + b_hh_b[:2 * H])).reshape(1, -1)
    bihbn = b_ih_b[2 * H:].reshape(1, -1)
    bhhbn = b_hh_b[2 * H:].reshape(1, -1)

    fwd_w = (wih2, whh_s, brz, bihn, bhhn)
    h = jnp.zeros((B2, 2 * H), jnp.float32)
    for c in range(NCH - 1):
        (h,) = _rnn_mid(xs[c], h, *fwd_w)
    mu, lv = _rnn_fin(
        xs[NCH - 1], h, *fwd_w,
        wihb2, brzb, bihbn, bhhbn,
        W_mu.T, b_mu.reshape(1, -1), W_lv.T, b_lv.reshape(1, -1),
    )
    return (mu, lv)


# speedup vs baseline: 1.3614x; 1.0200x over previous
import functools

import jax
import jax.numpy as jnp
from jax import lax
from jax.experimental import pallas as pl
from jax.experimental.pallas import tpu as pltpu
from jax.experimental.pallas import tpu_sc as plsc

V = 100000
E = 64
H = 256
L = 64
B = 1024
T = 200
B2 = B // 2

NCH = 5
TCH = T // NCH

NC = 2
NS = 16
NW = NC * NS
CHUNK = 128
NROWS = (T * B) // CHUNK
ROWS_C = NROWS // NCH
ROWS_W = ROWS_C // NW
OUT_C = TCH * B2


@functools.cache
def _make_sc_gather():
    mesh = plsc.VectorSubcoreMesh(
        core_axis_name="c", subcore_axis_name="s", num_cores=NC, num_subcores=NS
    )

    @functools.partial(
        pl.kernel,
        out_type=jax.ShapeDtypeStruct((OUT_C, 2 * E), jnp.float32),
        mesh=mesh,
        scratch_types=[
            pltpu.VMEM((ROWS_W, CHUNK), jnp.int32),
            pltpu.VMEM((CHUNK // 2, E), jnp.float32),
            pltpu.VMEM((CHUNK // 2, E), jnp.float32),
            pltpu.SemaphoreType.DMA,
        ],
        compiler_params=pltpu.CompilerParams(use_tc_tiling_on_sc=False),
    )
    def _sc_gather(emb_hbm, idx_hbm, out_hbm, idx_v, rows_a, rows_b, sem):
        wid = lax.axis_index("s") * NC + lax.axis_index("c")
        pltpu.sync_copy(idx_hbm.at[wid], idx_v)
        base2 = wid * ROWS_W * (CHUNK // 2)

        def body(j, carry):
            ca = pltpu.async_copy(emb_hbm.at[idx_v.at[j, pl.ds(0, CHUNK // 2)]],
                                  rows_a, sem)
            cb = pltpu.async_copy(emb_hbm.at[idx_v.at[j, pl.ds(CHUNK // 2, CHUNK // 2)]],
                                  rows_b, sem)
            ca.wait()
            cb.wait()
            row0 = base2 + j * (CHUNK // 2)
            pltpu.sync_copy(rows_a,
                            out_hbm.at[pl.ds(row0, CHUNK // 2), pl.ds(0, E)])
            pltpu.sync_copy(rows_b,
                            out_hbm.at[pl.ds(row0, CHUNK // 2), pl.ds(E, E)])
            return carry

        lax.fori_loop(0, ROWS_W, body, 0)

    return _sc_gather


UNROLL = 2


def _gru_steps(t, x_ref, hin_ref, wih2_ref, whh_ref, brz_ref, bihn_ref,
               bhhn_ref, h_scr, h16_scr):
    @pl.when(t == 0)
    def _():
        h_scr[...] = hin_ref[...]
        h16_scr[...] = hin_ref[...].astype(jnp.bfloat16)

    xps = x_ref[...]
    gi2u = jnp.dot(xps.reshape(UNROLL * B2, 2 * E).astype(jnp.bfloat16),
                   wih2_ref[...],
                   preferred_element_type=jnp.float32)
    hn_halves = None
    for u in range(UNROLL):
        hn_halves = []
        for c in range(2):
            h = h_scr[:, c * H:(c + 1) * H]
            gi = gi2u[u * B2:(u + 1) * B2, c * 3 * H:(c + 1) * 3 * H]
            gh = jnp.dot(h16_scr[:, c * H:(c + 1) * H], whh_ref[...],
                         preferred_element_type=jnp.float32)
            grz = gi[:, :2 * H] + gh[:, :2 * H] + brz_ref[...]
            r = 0.5 + 0.5 * jnp.tanh(grz[:, :H])
            z = 0.5 + 0.5 * jnp.tanh(grz[:, H:])
            n = jnp.tanh((gi[:, 2 * H:] + bihn_ref[...])
                         + r * (gh[:, 2 * H:] + bhhn_ref[...]))
            hn = n + z * (h - n)
            h_scr[:, c * H:(c + 1) * H] = hn
            h16_scr[:, c * H:(c + 1) * H] = hn.astype(jnp.bfloat16)
            hn_halves.append(hn)
    return xps[UNROLL - 1], hn_halves


def _rnn_mid_body(x_ref, hin_ref, wih2_ref, whh_ref, brz_ref, bihn_ref,
                  bhhn_ref, hout_ref, h_scr, h16_scr):
    t = pl.program_id(0)
    _gru_steps(t, x_ref, hin_ref, wih2_ref, whh_ref, brz_ref, bihn_ref,
               bhhn_ref, h_scr, h16_scr)

    @pl.when(t == TCH // UNROLL - 1)
    def _():
        hout_ref[...] = h_scr[...]


def _rnn_fin_body(x_ref, hin_ref, wih2_ref, whh_ref, brz_ref, bihn_ref,
                  bhhn_ref, wihb2_ref, brzb_ref, bihbn_ref, bhhbn_ref,
                  wmu_ref, bmu_ref, wlv_ref, blv_ref,
                  mu_ref, lv_ref, h_scr, h16_scr):
    t = pl.program_id(0)
    xp, hn_halves = _gru_steps(t, x_ref, hin_ref, wih2_ref, whh_ref, brz_ref,
                               bihn_ref, bhhn_ref, h_scr, h16_scr)

    @pl.when(t == TCH // UNROLL - 1)
    def _():
        gib2 = jnp.dot(xp.astype(jnp.bfloat16), wihb2_ref[...],
                       preferred_element_type=jnp.float32)
        for c in range(2):
            gib = gib2[:, c * 3 * H:(c + 1) * 3 * H]
            grzb = gib[:, :2 * H] + brzb_ref[...]
            rb = 0.5 + 0.5 * jnp.tanh(grzb[:, :H])
            zb = 0.5 - 0.5 * jnp.tanh(grzb[:, H:])
            nb = jnp.tanh((gib[:, 2 * H:] + bihbn_ref[...]) + rb * bhhbn_ref[...])
            out = hn_halves[c] + zb * nb
            mu_ref[c * B2:(c + 1) * B2, :] = (
                jnp.dot(out, wmu_ref[...], preferred_element_type=jnp.float32)
                + bmu_ref[...])
            lv_ref[c * B2:(c + 1) * B2, :] = (
                jnp.dot(out, wlv_ref[...], preferred_element_type=jnp.float32)
                + blv_ref[...])


_FULL2 = lambda t: (0, 0)

_X_SPEC = pl.BlockSpec((UNROLL, B2, 2 * E), lambda t: (t, 0, 0))
_H_SPEC = pl.BlockSpec((B2, 2 * H), _FULL2)
_FWD_W_SPECS = [
    _H_SPEC,
    pl.BlockSpec((2 * E, 6 * H), _FULL2),
    pl.BlockSpec((H, 3 * H), _FULL2),
    pl.BlockSpec((1, 2 * H), _FULL2),
    pl.BlockSpec((1, H), _FULL2),
    pl.BlockSpec((1, H), _FULL2),
]

_rnn_mid = pl.pallas_call(
    _rnn_mid_body,
    grid=(TCH // UNROLL,),
    in_specs=[_X_SPEC] + _FWD_W_SPECS,
    out_specs=[_H_SPEC],
    out_shape=[jax.ShapeDtypeStruct((B2, 2 * H), jnp.float32)],
    scratch_shapes=[pltpu.VMEM((B2, 2 * H), jnp.float32),
                    pltpu.VMEM((B2, 2 * H), jnp.bfloat16)],
)

_rnn_fin = pl.pallas_call(
    _rnn_fin_body,
    grid=(TCH // UNROLL,),
    in_specs=[_X_SPEC] + _FWD_W_SPECS + [
        pl.BlockSpec((2 * E, 6 * H), _FULL2),
        pl.BlockSpec((1, 2 * H), _FULL2),
        pl.BlockSpec((1, H), _FULL2),
        pl.BlockSpec((1, H), _FULL2),
        pl.BlockSpec((H, L), _FULL2),
        pl.BlockSpec((1, L), _FULL2),
        pl.BlockSpec((H, L), _FULL2),
        pl.BlockSpec((1, L), _FULL2),
    ],
    out_specs=[pl.BlockSpec((B, L), _FULL2), pl.BlockSpec((B, L), _FULL2)],
    out_shape=[jax.ShapeDtypeStruct((B, L), jnp.float32)] * 2,
    scratch_shapes=[pltpu.VMEM((B2, 2 * H), jnp.float32),
                    pltpu.VMEM((B2, 2 * H), jnp.bfloat16)],
)


def _blockdiag2(w):
    zero = jnp.zeros_like(w)
    return jnp.concatenate(
        [jnp.concatenate([w, zero], axis=1),
         jnp.concatenate([zero, w], axis=1)], axis=0)


def kernel(inputs, emb, W_ih_f, W_hh_f, b_ih_f, b_hh_f,
           W_ih_b, W_hh_b, b_ih_b, b_hh_b, W_mu, b_mu, W_lv, b_lv):
    ii = inputs.astype(jnp.int32)
    idx_a = ii[:B2].T.reshape(NROWS, CHUNK // 2)
    idx_b = ii[B2:].T.reshape(NROWS, CHUNK // 2)
    idx = jnp.concatenate([idx_a, idx_b], axis=1)

    gather = _make_sc_gather()
    xs = []
    for c in range(NCH):
        idx_c = idx[c * ROWS_C:(c + 1) * ROWS_C].reshape(NW, ROWS_W, CHUNK)
        xs.append(gather(emb, idx_c).reshape(TCH, B2, 2 * E))

    scale = jnp.concatenate(
        [jnp.full((2 * H,), 0.5, jnp.float32), jnp.ones((H,), jnp.float32)])
    wih2 = _blockdiag2(W_ih_f.T * scale).astype(jnp.bfloat16)
    whh_s = (W_hh_f.T * scale).astype(jnp.bfloat16)
    brz = (0.5 * (b_ih_f[:2 * H] + b_hh_f[:2 * H])).reshape(1, -1)
    bihn = b_ih_f[2 * H:].reshape(1, -1)
    bhhn = b_hh_f[2 * H:].reshape(1, -1)
    wihb2 = _blockdiag2(W_ih_b.T * scale).astype(jnp.bfloat16)
    brzb = (0.5 * (b_ih_b[:2 * H] + b_hh_b[:2 * H])).reshape(1, -1)
    bihbn = b_ih_b[2 * H:].reshape(1, -1)
    bhhbn = b_hh_b[2 * H:].reshape(1, -1)

    fwd_w = (wih2, whh_s, brz, bihn, bhhn)
    h = jnp.zeros((B2, 2 * H), jnp.float32)
    for c in range(NCH - 1):
        (h,) = _rnn_mid(xs[c], h, *fwd_w)
    mu, lv = _rnn_fin(
        xs[NCH - 1], h, *fwd_w,
        wihb2, brzb, bihbn, bhhbn,
        W_mu.T, b_mu.reshape(1, -1), W_lv.T, b_lv.reshape(1, -1),
    )
    return (mu, lv)


# speedup vs baseline: 1.3663x; 1.0036x over previous
import functools

import jax
import jax.numpy as jnp
from jax import lax
from jax.experimental import pallas as pl
from jax.experimental.pallas import tpu as pltpu
from jax.experimental.pallas import tpu_sc as plsc

V = 100000
E = 64
H = 256
L = 64
B = 1024
T = 200
B2 = B // 2

NCH = 5
TCH = T // NCH

NC = 2
NS = 16
NW = NC * NS
CHUNK = 128
NROWS = (T * B) // CHUNK
ROWS_C = NROWS // NCH
ROWS_W = ROWS_C // NW
OUT_C = TCH * B2


@functools.cache
def _make_sc_gather():
    mesh = plsc.VectorSubcoreMesh(
        core_axis_name="c", subcore_axis_name="s", num_cores=NC, num_subcores=NS
    )

    @functools.partial(
        pl.kernel,
        out_type=jax.ShapeDtypeStruct((OUT_C, 2 * E), jnp.float32),
        mesh=mesh,
        scratch_types=[
            pltpu.VMEM((ROWS_W, CHUNK), jnp.int32),
            pltpu.VMEM((CHUNK // 2, E), jnp.float32),
            pltpu.VMEM((CHUNK // 2, E), jnp.float32),
            pltpu.SemaphoreType.DMA,
        ],
        compiler_params=pltpu.CompilerParams(use_tc_tiling_on_sc=False),
    )
    def _sc_gather(emb_hbm, idx_hbm, out_hbm, idx_v, rows_a, rows_b, sem):
        wid = lax.axis_index("s") * NC + lax.axis_index("c")
        pltpu.sync_copy(idx_hbm.at[wid], idx_v)
        base2 = wid * ROWS_W * (CHUNK // 2)

        def body(j, carry):
            ca = pltpu.async_copy(emb_hbm.at[idx_v.at[j, pl.ds(0, CHUNK // 2)]],
                                  rows_a, sem)
            cb = pltpu.async_copy(emb_hbm.at[idx_v.at[j, pl.ds(CHUNK // 2, CHUNK // 2)]],
                                  rows_b, sem)
            ca.wait()
            cb.wait()
            row0 = base2 + j * (CHUNK // 2)
            pltpu.sync_copy(rows_a,
                            out_hbm.at[pl.ds(row0, CHUNK // 2), pl.ds(0, E)])
            pltpu.sync_copy(rows_b,
                            out_hbm.at[pl.ds(row0, CHUNK // 2), pl.ds(E, E)])
            return carry

        lax.fori_loop(0, ROWS_W, body, 0)

    return _sc_gather


UNROLL = 4


def _gru_steps(t, x_ref, hin_ref, wih2_ref, whh_ref, brz_ref, bihn_ref,
               bhhn_ref, h_scr, h16_scr):
    @pl.when(t == 0)
    def _():
        h_scr[...] = hin_ref[...]
        h16_scr[...] = hin_ref[...].astype(jnp.bfloat16)

    xps = x_ref[...]
    gi2u = jnp.dot(xps.reshape(UNROLL * B2, 2 * E).astype(jnp.bfloat16),
                   wih2_ref[...],
                   preferred_element_type=jnp.float32)
    hn_halves = None
    for u in range(UNROLL):
        hn_halves = []
        for c in range(2):
            h = h_scr[:, c * H:(c + 1) * H]
            gi = gi2u[u * B2:(u + 1) * B2, c * 3 * H:(c + 1) * 3 * H]
            gh = jnp.dot(h16_scr[:, c * H:(c + 1) * H], whh_ref[...],
                         preferred_element_type=jnp.float32)
            grz = gi[:, :2 * H] + gh[:, :2 * H] + brz_ref[...]
            r = 0.5 + 0.5 * jnp.tanh(grz[:, :H])
            z = 0.5 + 0.5 * jnp.tanh(grz[:, H:])
            n = jnp.tanh((gi[:, 2 * H:] + bihn_ref[...])
                         + r * (gh[:, 2 * H:] + bhhn_ref[...]))
            hn = n + z * (h - n)
            h_scr[:, c * H:(c + 1) * H] = hn
            h16_scr[:, c * H:(c + 1) * H] = hn.astype(jnp.bfloat16)
            hn_halves.append(hn)
    return xps[UNROLL - 1], hn_halves


def _rnn_mid_body(x_ref, hin_ref, wih2_ref, whh_ref, brz_ref, bihn_ref,
                  bhhn_ref, hout_ref, h_scr, h16_scr):
    t = pl.program_id(0)
    _gru_steps(t, x_ref, hin_ref, wih2_ref, whh_ref, brz_ref, bihn_ref,
               bhhn_ref, h_scr, h16_scr)

    @pl.when(t == TCH // UNROLL - 1)
    def _():
        hout_ref[...] = h_scr[...]


def _rnn_fin_body(x_ref, hin_ref, wih2_ref, whh_ref, brz_ref, bihn_ref,
                  bhhn_ref, wihb2_ref, brzb_ref, bihbn_ref, bhhbn_ref,
                  wmu_ref, bmu_ref, wlv_ref, blv_ref,
                  mu_ref, lv_ref, h_scr, h16_scr):
    t = pl.program_id(0)
    xp, hn_halves = _gru_steps(t, x_ref, hin_ref, wih2_ref, whh_ref, brz_ref,
                               bihn_ref, bhhn_ref, h_scr, h16_scr)

    @pl.when(t == TCH // UNROLL - 1)
    def _():
        gib2 = jnp.dot(xp.astype(jnp.bfloat16), wihb2_ref[...],
                       preferred_element_type=jnp.float32)
        for c in range(2):
            gib = gib2[:, c * 3 * H:(c + 1) * 3 * H]
            grzb = gib[:, :2 * H] + brzb_ref[...]
            rb = 0.5 + 0.5 * jnp.tanh(grzb[:, :H])
            zb = 0.5 - 0.5 * jnp.tanh(grzb[:, H:])
            nb = jnp.tanh((gib[:, 2 * H:] + bihbn_ref[...]) + rb * bhhbn_ref[...])
            out = hn_halves[c] + zb * nb
            mu_ref[c * B2:(c + 1) * B2, :] = (
                jnp.dot(out, wmu_ref[...], preferred_element_type=jnp.float32)
                + bmu_ref[...])
            lv_ref[c * B2:(c + 1) * B2, :] = (
                jnp.dot(out, wlv_ref[...], preferred_element_type=jnp.float32)
                + blv_ref[...])


_FULL2 = lambda t: (0, 0)

_X_SPEC = pl.BlockSpec((UNROLL, B2, 2 * E), lambda t: (t, 0, 0))
_H_SPEC = pl.BlockSpec((B2, 2 * H), _FULL2)
_FWD_W_SPECS = [
    _H_SPEC,
    pl.BlockSpec((2 * E, 6 * H), _FULL2),
    pl.BlockSpec((H, 3 * H), _FULL2),
    pl.BlockSpec((1, 2 * H), _FULL2),
    pl.BlockSpec((1, H), _FULL2),
    pl.BlockSpec((1, H), _FULL2),
]

_rnn_mid = pl.pallas_call(
    _rnn_mid_body,
    grid=(TCH // UNROLL,),
    in_specs=[_X_SPEC] + _FWD_W_SPECS,
    out_specs=[_H_SPEC],
    out_shape=[jax.ShapeDtypeStruct((B2, 2 * H), jnp.float32)],
    scratch_shapes=[pltpu.VMEM((B2, 2 * H), jnp.float32),
                    pltpu.VMEM((B2, 2 * H), jnp.bfloat16)],
)

_rnn_fin = pl.pallas_call(
    _rnn_fin_body,
    grid=(TCH // UNROLL,),
    in_specs=[_X_SPEC] + _FWD_W_SPECS + [
        pl.BlockSpec((2 * E, 6 * H), _FULL2),
        pl.BlockSpec((1, 2 * H), _FULL2),
        pl.BlockSpec((1, H), _FULL2),
        pl.BlockSpec((1, H), _FULL2),
        pl.BlockSpec((H, L), _FULL2),
        pl.BlockSpec((1, L), _FULL2),
        pl.BlockSpec((H, L), _FULL2),
        pl.BlockSpec((1, L), _FULL2),
    ],
    out_specs=[pl.BlockSpec((B, L), _FULL2), pl.BlockSpec((B, L), _FULL2)],
    out_shape=[jax.ShapeDtypeStruct((B, L), jnp.float32)] * 2,
    scratch_shapes=[pltpu.VMEM((B2, 2 * H), jnp.float32),
                    pltpu.VMEM((B2, 2 * H), jnp.bfloat16)],
)


def _blockdiag2(w):
    zero = jnp.zeros_like(w)
    return jnp.concatenate(
        [jnp.concatenate([w, zero], axis=1),
         jnp.concatenate([zero, w], axis=1)], axis=0)


def kernel(inputs, emb, W_ih_f, W_hh_f, b_ih_f, b_hh_f,
           W_ih_b, W_hh_b, b_ih_b, b_hh_b, W_mu, b_mu, W_lv, b_lv):
    ii = inputs.astype(jnp.int32)
    idx_a = ii[:B2].T.reshape(NROWS, CHUNK // 2)
    idx_b = ii[B2:].T.reshape(NROWS, CHUNK // 2)
    idx = jnp.concatenate([idx_a, idx_b], axis=1)

    gather = _make_sc_gather()
    xs = []
    for c in range(NCH):
        idx_c = idx[c * ROWS_C:(c + 1) * ROWS_C].reshape(NW, ROWS_W, CHUNK)
        xs.append(gather(emb, idx_c).reshape(TCH, B2, 2 * E))

    scale = jnp.concatenate(
        [jnp.full((2 * H,), 0.5, jnp.float32), jnp.ones((H,), jnp.float32)])
    wih2 = _blockdiag2(W_ih_f.T * scale).astype(jnp.bfloat16)
    whh_s = (W_hh_f.T * scale).astype(jnp.bfloat16)
    brz = (0.5 * (b_ih_f[:2 * H] + b_hh_f[:2 * H])).reshape(1, -1)
    bihn = b_ih_f[2 * H:].reshape(1, -1)
    bhhn = b_hh_f[2 * H:].reshape(1, -1)
    wihb2 = _blockdiag2(W_ih_b.T * scale).astype(jnp.bfloat16)
    brzb = (0.5 * (b_ih_b[:2 * H] + b_hh_b[:2 * H])).reshape(1, -1)
    bihbn = b_ih_b[2 * H:].reshape(1, -1)
    bhhbn = b_hh_b[2 * H:].reshape(1, -1)

    fwd_w = (wih2, whh_s, brz, bihn, bhhn)
    h = jnp.zeros((B2, 2 * H), jnp.float32)
    for c in range(NCH - 1):
        (h,) = _rnn_mid(xs[c], h, *fwd_w)
    mu, lv = _rnn_fin(
        xs[NCH - 1], h, *fwd_w,
        wihb2, brzb, bihbn, bhhbn,
        W_mu.T, b_mu.reshape(1, -1), W_lv.T, b_lv.reshape(1, -1),
    )
    return (mu, lv)


# speedup vs baseline: 1.3792x; 1.0094x over previous
import functools

import jax
import jax.numpy as jnp
from jax import lax
from jax.experimental import pallas as pl
from jax.experimental.pallas import tpu as pltpu
from jax.experimental.pallas import tpu_sc as plsc

V = 100000
E = 64
H = 256
L = 64
B = 1024
T = 200
B2 = B // 2

CH_SIZES = (8, 48, 48, 48, 48)

NC = 2
NS = 16
NW = NC * NS
CHUNK = 128


@functools.cache
def _make_sc_gather(tch):
    rows_w = tch * B * 2 // CHUNK // 2 // NW
    out_c = tch * B2
    mesh = plsc.VectorSubcoreMesh(
        core_axis_name="c", subcore_axis_name="s", num_cores=NC, num_subcores=NS
    )

    @functools.partial(
        pl.kernel,
        out_type=jax.ShapeDtypeStruct((out_c, 2 * E), jnp.float32),
        mesh=mesh,
        scratch_types=[
            pltpu.VMEM((rows_w, CHUNK), jnp.int32),
            pltpu.VMEM((CHUNK // 2, E), jnp.float32),
            pltpu.VMEM((CHUNK // 2, E), jnp.float32),
            pltpu.SemaphoreType.DMA,
        ],
        compiler_params=pltpu.CompilerParams(use_tc_tiling_on_sc=False),
    )
    def _sc_gather(emb_hbm, idx_hbm, out_hbm, idx_v, rows_a, rows_b, sem):
        wid = lax.axis_index("s") * NC + lax.axis_index("c")
        pltpu.sync_copy(idx_hbm.at[wid], idx_v)
        base2 = wid * rows_w * (CHUNK // 2)

        def body(j, carry):
            ca = pltpu.async_copy(emb_hbm.at[idx_v.at[j, pl.ds(0, CHUNK // 2)]],
                                  rows_a, sem)
            cb = pltpu.async_copy(emb_hbm.at[idx_v.at[j, pl.ds(CHUNK // 2, CHUNK // 2)]],
                                  rows_b, sem)
            ca.wait()
            cb.wait()
            row0 = base2 + j * (CHUNK // 2)
            pltpu.sync_copy(rows_a,
                            out_hbm.at[pl.ds(row0, CHUNK // 2), pl.ds(0, E)])
            pltpu.sync_copy(rows_b,
                            out_hbm.at[pl.ds(row0, CHUNK // 2), pl.ds(E, E)])
            return carry

        lax.fori_loop(0, rows_w, body, 0)

    return _sc_gather


UNROLL = 4


def _gru_steps(t, x_ref, hin_ref, wih2_ref, whh_ref, brz_ref, bihn_ref,
               bhhn_ref, h_scr, h16_scr):
    @pl.when(t == 0)
    def _():
        h_scr[...] = hin_ref[...]
        h16_scr[...] = hin_ref[...].astype(jnp.bfloat16)

    xps = x_ref[...]
    gi2u = jnp.dot(xps.reshape(UNROLL * B2, 2 * E).astype(jnp.bfloat16),
                   wih2_ref[...],
                   preferred_element_type=jnp.float32)
    hn_halves = None
    for u in range(UNROLL):
        hn_halves = []
        for c in range(2):
            h = h_scr[:, c * H:(c + 1) * H]
            gi = gi2u[u * B2:(u + 1) * B2, c * 3 * H:(c + 1) * 3 * H]
            gh = jnp.dot(h16_scr[:, c * H:(c + 1) * H], whh_ref[...],
                         preferred_element_type=jnp.float32)
            grz = gi[:, :2 * H] + gh[:, :2 * H] + brz_ref[...]
            r = 0.5 + 0.5 * jnp.tanh(grz[:, :H])
            z = 0.5 + 0.5 * jnp.tanh(grz[:, H:])
            n = jnp.tanh((gi[:, 2 * H:] + bihn_ref[...])
                         + r * (gh[:, 2 * H:] + bhhn_ref[...]))
            hn = n + z * (h - n)
            h_scr[:, c * H:(c + 1) * H] = hn
            h16_scr[:, c * H:(c + 1) * H] = hn.astype(jnp.bfloat16)
            hn_halves.append(hn)
    return xps[UNROLL - 1], hn_halves


def _make_mid_body(tch):
  def _rnn_mid_body(x_ref, hin_ref, wih2_ref, whh_ref, brz_ref, bihn_ref,
                    bhhn_ref, hout_ref, h_scr, h16_scr):
    t = pl.program_id(0)
    _gru_steps(t, x_ref, hin_ref, wih2_ref, whh_ref, brz_ref, bihn_ref,
               bhhn_ref, h_scr, h16_scr)

    @pl.when(t == tch // UNROLL - 1)
    def _():
        hout_ref[...] = h_scr[...]
  return _rnn_mid_body


def _make_fin_body(tch):
  def _rnn_fin_body(x_ref, hin_ref, wih2_ref, whh_ref, brz_ref, bihn_ref,
                  bhhn_ref, wihb2_ref, brzb_ref, bihbn_ref, bhhbn_ref,
                  wmu_ref, bmu_ref, wlv_ref, blv_ref,
                  mu_ref, lv_ref, h_scr, h16_scr):
    t = pl.program_id(0)
    xp, hn_halves = _gru_steps(t, x_ref, hin_ref, wih2_ref, whh_ref, brz_ref,
                               bihn_ref, bhhn_ref, h_scr, h16_scr)

    @pl.when(t == tch // UNROLL - 1)
    def _():
        gib2 = jnp.dot(xp.astype(jnp.bfloat16), wihb2_ref[...],
                       preferred_element_type=jnp.float32)
        for c in range(2):
            gib = gib2[:, c * 3 * H:(c + 1) * 3 * H]
            grzb = gib[:, :2 * H] + brzb_ref[...]
            rb = 0.5 + 0.5 * jnp.tanh(grzb[:, :H])
            zb = 0.5 - 0.5 * jnp.tanh(grzb[:, H:])
            nb = jnp.tanh((gib[:, 2 * H:] + bihbn_ref[...]) + rb * bhhbn_ref[...])
            out = hn_halves[c] + zb * nb
            mu_ref[c * B2:(c + 1) * B2, :] = (
                jnp.dot(out, wmu_ref[...], preferred_element_type=jnp.float32)
                + bmu_ref[...])
            lv_ref[c * B2:(c + 1) * B2, :] = (
                jnp.dot(out, wlv_ref[...], preferred_element_type=jnp.float32)
                + blv_ref[...])
  return _rnn_fin_body


_FULL2 = lambda t: (0, 0)

_H_SPEC = pl.BlockSpec((B2, 2 * H), _FULL2)
_FWD_W_SPECS = [
    _H_SPEC,
    pl.BlockSpec((2 * E, 6 * H), _FULL2),
    pl.BlockSpec((H, 3 * H), _FULL2),
    pl.BlockSpec((1, 2 * H), _FULL2),
    pl.BlockSpec((1, H), _FULL2),
    pl.BlockSpec((1, H), _FULL2),
]


def _x_spec():
    return pl.BlockSpec((UNROLL, B2, 2 * E), lambda t: (t, 0, 0))


@functools.cache
def _make_rnn_mid(tch):
    return pl.pallas_call(
        _make_mid_body(tch),
        grid=(tch // UNROLL,),
        in_specs=[_x_spec()] + _FWD_W_SPECS,
        out_specs=[_H_SPEC],
        out_shape=[jax.ShapeDtypeStruct((B2, 2 * H), jnp.float32)],
        scratch_shapes=[pltpu.VMEM((B2, 2 * H), jnp.float32),
                        pltpu.VMEM((B2, 2 * H), jnp.bfloat16)],
    )


@functools.cache
def _make_rnn_fin(tch):
    return pl.pallas_call(
        _make_fin_body(tch),
        grid=(tch // UNROLL,),
        in_specs=[_x_spec()] + _FWD_W_SPECS + [
        pl.BlockSpec((2 * E, 6 * H), _FULL2),
        pl.BlockSpec((1, 2 * H), _FULL2),
        pl.BlockSpec((1, H), _FULL2),
        pl.BlockSpec((1, H), _FULL2),
        pl.BlockSpec((H, L), _FULL2),
        pl.BlockSpec((1, L), _FULL2),
        pl.BlockSpec((H, L), _FULL2),
        pl.BlockSpec((1, L), _FULL2),
    ],
        out_specs=[pl.BlockSpec((B, L), _FULL2), pl.BlockSpec((B, L), _FULL2)],
        out_shape=[jax.ShapeDtypeStruct((B, L), jnp.float32)] * 2,
        scratch_shapes=[pltpu.VMEM((B2, 2 * H), jnp.float32),
                        pltpu.VMEM((B2, 2 * H), jnp.bfloat16)],
    )


def _blockdiag2(w):
    zero = jnp.zeros_like(w)
    return jnp.concatenate(
        [jnp.concatenate([w, zero], axis=1),
         jnp.concatenate([zero, w], axis=1)], axis=0)


def kernel(inputs, emb, W_ih_f, W_hh_f, b_ih_f, b_hh_f,
           W_ih_b, W_hh_b, b_ih_b, b_hh_b, W_mu, b_mu, W_lv, b_lv):
    ii = inputs.astype(jnp.int32)

    xs = []
    t0 = 0
    for tch in CH_SIZES:
        rows_c = tch * B // CHUNK
        rows_w = rows_c // NW
        ia = ii[:B2, t0:t0 + tch].T.reshape(rows_c, CHUNK // 2)
        ib = ii[B2:, t0:t0 + tch].T.reshape(rows_c, CHUNK // 2)
        idx_c = jnp.concatenate([ia, ib], axis=1).reshape(NW, rows_w, CHUNK)
        xs.append(_make_sc_gather(tch)(emb, idx_c).reshape(tch, B2, 2 * E))
        t0 += tch

    scale = jnp.concatenate(
        [jnp.full((2 * H,), 0.5, jnp.float32), jnp.ones((H,), jnp.float32)])
    wih2 = _blockdiag2(W_ih_f.T * scale).astype(jnp.bfloat16)
    whh_s = (W_hh_f.T * scale).astype(jnp.bfloat16)
    brz = (0.5 * (b_ih_f[:2 * H] + b_hh_f[:2 * H])).reshape(1, -1)
    bihn = b_ih_f[2 * H:].reshape(1, -1)
    bhhn = b_hh_f[2 * H:].reshape(1, -1)
    wihb2 = _blockdiag2(W_ih_b.T * scale).astype(jnp.bfloat16)
    brzb = (0.5 * (b_ih_b[:2 * H] + b_hh_b[:2 * H])).reshape(1, -1)
    bihbn = b_ih_b[2 * H:].reshape(1, -1)
    bhhbn = b_hh_b[2 * H:].reshape(1, -1)

    fwd_w = (wih2, whh_s, brz, bihn, bhhn)
    h = jnp.zeros((B2, 2 * H), jnp.float32)
    for c in range(len(CH_SIZES) - 1):
        (h,) = _make_rnn_mid(CH_SIZES[c])(xs[c], h, *fwd_w)
    mu, lv = _make_rnn_fin(CH_SIZES[-1])(
        xs[-1], h, *fwd_w,
        wihb2, brzb, bihbn, bhhbn,
        W_mu.T, b_mu.reshape(1, -1), W_lv.T, b_lv.reshape(1, -1),
    )
    return (mu, lv)
